# baseline (speedup 1.0000x reference)
import functools

import jax
import jax.numpy as jnp
from jax import lax
from jax.experimental import pallas as pl
from jax.experimental.pallas import tpu as pltpu
from jax.experimental.pallas import tpu_sc as plsc

N_NODES = 10000
C = 256
NUM_CORES = 2
NUM_SUBCORES = 16
NW = NUM_CORES * NUM_SUBCORES
R = 320
PAD_N = NW * R
STRIP = 6400
GK = 32


def _sc_aggregate(x, src, dst):
    E = src.shape[0]
    n_strips = E // STRIP
    n_cchunks = STRIP // 16

    mesh = plsc.VectorSubcoreMesh(
        core_axis_name="c", subcore_axis_name="s",
        num_cores=NUM_CORES, num_subcores=NUM_SUBCORES)

    @functools.partial(
        pl.kernel,
        out_type=(jax.ShapeDtypeStruct((PAD_N, C), jnp.float32),
                  jax.ShapeDtypeStruct((NW * 8, C), jnp.float32)),
        mesh=mesh,
        compiler_params=pltpu.CompilerParams(needs_layout_passes=False),
        scratch_types=[
            pltpu.VMEM((STRIP,), jnp.int32),
            pltpu.VMEM((STRIP,), jnp.int32),
            pltpu.VMEM((STRIP + 32,), jnp.int32),
            pltpu.VMEM((STRIP + 32,), jnp.int32),
            pltpu.VMEM((GK, C), jnp.float32),
            pltpu.VMEM((GK, C), jnp.float32),
            pltpu.SemaphoreType.DMA,
            pltpu.SemaphoreType.DMA,
            pltpu.VMEM((R, C), jnp.float32),
            pltpu.VMEM((R + 16,), jnp.float32),
            pltpu.VMEM((8, C), jnp.float32),
        ],
    )
    def k(x_hbm, src_hbm, dst_hbm, agg_out, deg_out,
          ssrc_v, sdst_v, psrc_v, pdl_v, rows_a, rows_b, sem_a, sem_b,
          acc_v, deg_v, dst_stage_v):
        cid = lax.axis_index("c")
        sid = lax.axis_index("s")
        wid = cid * NUM_SUBCORES + sid
        base = wid * R

        zero16 = jnp.zeros((16,), jnp.float32)
        one16 = jnp.ones((16,), jnp.float32)
        lane0 = lax.iota(jnp.int32, 16) == 0

        @pl.loop(0, R)
        def _(i):
            for j in range(C // 16):
                acc_v[i, pl.ds(j * 16, 16)] = zero16

        @pl.loop(0, (R + 16) // 16)
        def _(i):
            deg_v[pl.ds(i * 16, 16)] = zero16

        @pl.loop(0, (STRIP + 32) // 16)
        def _(i):
            psrc_v[pl.ds(i * 16, 16)] = jnp.zeros((16,), jnp.int32)

        @pl.loop(0, n_strips)
        def _(s):
            e0 = s * STRIP
            pltpu.sync_copy(src_hbm.at[pl.ds(e0, STRIP)], ssrc_v)
            pltpu.sync_copy(dst_hbm.at[pl.ds(e0, STRIP)], sdst_v)

            def compress(j, off):
                dv = sdst_v[pl.ds(j * 16, 16)]
                sv = ssrc_v[pl.ds(j * 16, 16)]
                rel = dv - base
                m = rel.astype(jnp.uint32) < R
                plsc.store_compressed(psrc_v.at[pl.ds(off, 16)], sv, mask=m)
                plsc.store_compressed(pdl_v.at[pl.ds(off, 16)], rel, mask=m)
                return off + plsc.all_reduce_population_count(m)[0]

            kk = pl.loop(0, n_cchunks, init_carry=0, unroll=2)(compress)

            nch = (kk + GK - 1) // GK

            def fire(c, buf, sem):
                pltpu.make_async_copy(
                    x_hbm.at[psrc_v.at[pl.ds(c * GK, GK)]], buf, sem).start()

            def drain(c, buf, sem):
                pltpu.make_async_copy(
                    x_hbm.at[psrc_v.at[pl.ds(c * GK, GK)]], buf, sem).wait()
                nrows = jnp.minimum(GK, kk - c * GK)

                @pl.loop(0, nrows)
                def _(r):
                    dl = pdl_v[pl.ds(c * GK + r, 16)][0]
                    for j in range(C // 16):
                        plsc.addupdate(acc_v.at[dl, pl.ds(j * 16, 16)],
                                       buf[r, pl.ds(j * 16, 16)])
                    plsc.addupdate_scatter(
                        deg_v, [jnp.full((16,), dl, jnp.int32)], one16,
                        mask=lane0)

            @pl.when(nch > 0)
            def _():
                fire(0, rows_a, sem_a)

            @pl.loop(0, (nch + 1) // 2)
            def _(pair):
                c0 = 2 * pair
                c1 = c0 + 1

                @pl.when(c1 < nch)
                def _():
                    fire(c1, rows_b, sem_b)

                drain(c0, rows_a, sem_a)

                @pl.when(c1 + 1 < nch)
                def _():
                    fire(c1 + 1, rows_a, sem_a)

                @pl.when(c1 < nch)
                def _():
                    drain(c1, rows_b, sem_b)

        for j in range(16):
            dst_stage_v[0, pl.ds(j * 16, 16)] = deg_v[pl.ds(j * 16, 16)]
        for j in range(4):
            dst_stage_v[1, pl.ds(j * 16, 16)] = deg_v[pl.ds(256 + j * 16, 16)]
        pltpu.sync_copy(acc_v, agg_out.at[pl.ds(base, R)])
        pltpu.sync_copy(dst_stage_v, deg_out.at[pl.ds(wid * 8, 8)])

    return k(x, src, dst)


BLK = 1000


def _pre_body(x_ref, w_ref, o_ref):
    o_ref[...] = jnp.dot(x_ref[...], w_ref[...],
                         preferred_element_type=jnp.float32)


def _dense_pre(x, W_cat):
    n = x.shape[0]
    return pl.pallas_call(
        _pre_body,
        grid=(n // BLK,),
        in_specs=[
            pl.BlockSpec((BLK, C), lambda i: (i, 0)),
            pl.BlockSpec((C, 2 * C), lambda i: (0, 0)),
        ],
        out_specs=pl.BlockSpec((BLK, 2 * C), lambda i: (i, 0)),
        out_shape=jax.ShapeDtypeStruct((n, 2 * C), jnp.float32),
    )(x, W_cat)


def _post_body(w_ref, x_ref, a_ref, d_ref, p_ref, wgn_ref, wi_ref, o_ref):
    x_b = x_ref[...]
    s_b = a_ref[...]
    deg = jnp.maximum(d_ref[...][:, 0:1], 1.0)
    m_b = s_b / deg
    w1 = w_ref[1, 0]
    w2 = w_ref[2, 0]
    w3 = w_ref[3, 0]
    w4 = w_ref[4, 0]
    f32 = jnp.float32
    gs = jnp.dot(m_b, wgn_ref[...], preferred_element_type=f32)
    gcn = jax.nn.relu(gs[:, :C])
    sage = jax.nn.relu(p_ref[:, :C] + gs[:, C:])
    gin = jax.nn.relu(p_ref[:, C:]
                      + jnp.dot(s_b, wi_ref[...], preferred_element_type=f32))
    o_ref[...] = w1 * x_b + w2 * gcn + w3 * sage + w4 * gin


def _dense_post(x, agg, deg2d, pre, wvec, Wgn_cat, W_gin):
    n = x.shape[0]
    return pl.pallas_call(
        _post_body,
        grid=(n // BLK,),
        in_specs=[
            pl.BlockSpec((8, 128), lambda i: (0, 0)),
            pl.BlockSpec((BLK, C), lambda i: (i, 0)),
            pl.BlockSpec((BLK, C), lambda i: (i, 0)),
            pl.BlockSpec((BLK, 128), lambda i: (i, 0)),
            pl.BlockSpec((BLK, 2 * C), lambda i: (i, 0)),
            pl.BlockSpec((C, 2 * C), lambda i: (0, 0)),
            pl.BlockSpec((C, C), lambda i: (0, 0)),
        ],
        out_specs=pl.BlockSpec((BLK, C), lambda i: (i, 0)),
        out_shape=jax.ShapeDtypeStruct((n, C), jnp.float32),
    )(wvec, x, agg, deg2d, pre, Wgn_cat, W_gin)


def kernel(x, edge_index, weights, W_gcn, W_sage_self, W_sage_neigh, W_gin):
    src = edge_index[0]
    dst = edge_index[1]
    agg, deg_blk = _sc_aggregate(x, src, dst)
    pre = _dense_pre(x, jnp.concatenate([W_sage_self, W_gin], axis=1))
    deg = deg_blk.reshape(NW, 8 * C)[:, :R].reshape(PAD_N)
    deg2d = jnp.broadcast_to(deg[:N_NODES, None], (N_NODES, 128))
    wvec = jnp.pad(jnp.broadcast_to(weights.reshape(5, 1), (5, 128)),
                   ((0, 3), (0, 0)))
    return _dense_post(x, agg[:N_NODES], deg2d, pre, wvec,
                       jnp.concatenate([W_gcn, W_sage_neigh], axis=1), W_gin)


# speedup vs baseline: 1.4621x; 1.4621x over previous
import functools

import jax
import jax.numpy as jnp
from jax import lax
from jax.experimental import pallas as pl
from jax.experimental.pallas import tpu as pltpu
from jax.experimental.pallas import tpu_sc as plsc

N_NODES = 10000
C = 256
NUM_CORES = 2
NUM_SUBCORES = 16
NW = NUM_CORES * NUM_SUBCORES
R = 320
PAD_N = NW * R
STRIP = 6400
GK = 32


def _sc_aggregate(x, src, dst):
    E = src.shape[0]
    n_strips = E // STRIP
    n_cchunks = STRIP // 16

    mesh = plsc.VectorSubcoreMesh(
        core_axis_name="c", subcore_axis_name="s",
        num_cores=NUM_CORES, num_subcores=NUM_SUBCORES)

    @functools.partial(
        pl.kernel,
        out_type=(jax.ShapeDtypeStruct((PAD_N, C), jnp.float32),
                  jax.ShapeDtypeStruct((NW * 8, C), jnp.float32)),
        mesh=mesh,
        compiler_params=pltpu.CompilerParams(needs_layout_passes=False),
        scratch_types=[
            pltpu.VMEM((STRIP,), jnp.int32),
            pltpu.VMEM((STRIP,), jnp.int32),
            pltpu.VMEM((STRIP + 32,), jnp.int32),
            pltpu.VMEM((STRIP + 32,), jnp.int32),
            pltpu.VMEM((GK, C), jnp.float32),
            pltpu.VMEM((GK, C), jnp.float32),
            pltpu.SemaphoreType.DMA,
            pltpu.SemaphoreType.DMA,
            pltpu.VMEM((R, C), jnp.float32),
            pltpu.VMEM((R + 16,), jnp.float32),
            pltpu.VMEM((8, C), jnp.float32),
        ],
    )
    def k(x_hbm, src_hbm, dst_hbm, agg_out, deg_out,
          ssrc_v, sdst_v, psrc_v, pdl_v, rows_a, rows_b, sem_a, sem_b,
          acc_v, deg_v, dst_stage_v):
        cid = lax.axis_index("c")
        sid = lax.axis_index("s")
        wid = cid * NUM_SUBCORES + sid
        base = wid * R

        zero16 = jnp.zeros((16,), jnp.float32)
        one16 = jnp.ones((16,), jnp.float32)
        lane0 = lax.iota(jnp.int32, 16) == 0

        @pl.loop(0, R)
        def _(i):
            for j in range(C // 16):
                acc_v[i, pl.ds(j * 16, 16)] = zero16

        @pl.loop(0, (R + 16) // 16)
        def _(i):
            deg_v[pl.ds(i * 16, 16)] = zero16

        @pl.loop(0, (STRIP + 32) // 16)
        def _(i):
            psrc_v[pl.ds(i * 16, 16)] = jnp.zeros((16,), jnp.int32)

        @pl.loop(0, n_strips)
        def _(s):
            e0 = s * STRIP
            pltpu.sync_copy(src_hbm.at[pl.ds(e0, STRIP)], ssrc_v)
            pltpu.sync_copy(dst_hbm.at[pl.ds(e0, STRIP)], sdst_v)

            def compress(j, off):
                dv = sdst_v[pl.ds(j * 16, 16)]
                sv = ssrc_v[pl.ds(j * 16, 16)]
                rel = dv - base
                m = rel.astype(jnp.uint32) < R
                plsc.store_compressed(psrc_v.at[pl.ds(off, 16)], sv, mask=m)
                plsc.store_compressed(pdl_v.at[pl.ds(off, 16)], rel, mask=m)
                return off + plsc.all_reduce_population_count(m)[0]

            kk = pl.loop(0, n_cchunks, init_carry=0, unroll=2)(compress)

            nch = (kk + GK - 1) // GK

            def fire(c, buf, sem):
                pltpu.make_async_copy(
                    x_hbm.at[psrc_v.at[pl.ds(c * GK, GK)]], buf, sem).start()

            def drain(c, buf, sem):
                pltpu.make_async_copy(
                    x_hbm.at[psrc_v.at[pl.ds(c * GK, GK)]], buf, sem).wait()
                nrows = jnp.minimum(GK, kk - c * GK)

                @plsc.parallel_loop(0, nrows)
                def _(r):
                    dl = pdl_v[pl.ds(c * GK + r, 16)][0]
                    for j in range(C // 16):
                        plsc.addupdate(acc_v.at[dl, pl.ds(j * 16, 16)],
                                       buf[r, pl.ds(j * 16, 16)])
                    plsc.addupdate_scatter(
                        deg_v, [jnp.full((16,), dl, jnp.int32)], one16,
                        mask=lane0)

            @pl.when(nch > 0)
            def _():
                fire(0, rows_a, sem_a)

            @pl.loop(0, (nch + 1) // 2)
            def _(pair):
                c0 = 2 * pair
                c1 = c0 + 1

                @pl.when(c1 < nch)
                def _():
                    fire(c1, rows_b, sem_b)

                drain(c0, rows_a, sem_a)

                @pl.when(c1 + 1 < nch)
                def _():
                    fire(c1 + 1, rows_a, sem_a)

                @pl.when(c1 < nch)
                def _():
                    drain(c1, rows_b, sem_b)

        for j in range(16):
            dst_stage_v[0, pl.ds(j * 16, 16)] = deg_v[pl.ds(j * 16, 16)]
        for j in range(4):
            dst_stage_v[1, pl.ds(j * 16, 16)] = deg_v[pl.ds(256 + j * 16, 16)]
        pltpu.sync_copy(acc_v, agg_out.at[pl.ds(base, R)])
        pltpu.sync_copy(dst_stage_v, deg_out.at[pl.ds(wid * 8, 8)])

    return k(x, src, dst)


BLK = 1000


def _pre_body(x_ref, w_ref, o_ref):
    o_ref[...] = jnp.dot(x_ref[...], w_ref[...],
                         preferred_element_type=jnp.float32)


def _dense_pre(x, W_cat):
    n = x.shape[0]
    return pl.pallas_call(
        _pre_body,
        grid=(n // BLK,),
        in_specs=[
            pl.BlockSpec((BLK, C), lambda i: (i, 0)),
            pl.BlockSpec((C, 2 * C), lambda i: (0, 0)),
        ],
        out_specs=pl.BlockSpec((BLK, 2 * C), lambda i: (i, 0)),
        out_shape=jax.ShapeDtypeStruct((n, 2 * C), jnp.float32),
    )(x, W_cat)


def _post_body(w_ref, x_ref, a_ref, d_ref, p_ref, wgn_ref, wi_ref, o_ref):
    x_b = x_ref[...]
    s_b = a_ref[...]
    deg = jnp.maximum(d_ref[...][:, 0:1], 1.0)
    m_b = s_b / deg
    w1 = w_ref[1, 0]
    w2 = w_ref[2, 0]
    w3 = w_ref[3, 0]
    w4 = w_ref[4, 0]
    f32 = jnp.float32
    gs = jnp.dot(m_b, wgn_ref[...], preferred_element_type=f32)
    gcn = jax.nn.relu(gs[:, :C])
    sage = jax.nn.relu(p_ref[:, :C] + gs[:, C:])
    gin = jax.nn.relu(p_ref[:, C:]
                      + jnp.dot(s_b, wi_ref[...], preferred_element_type=f32))
    o_ref[...] = w1 * x_b + w2 * gcn + w3 * sage + w4 * gin


def _dense_post(x, agg, deg2d, pre, wvec, Wgn_cat, W_gin):
    n = x.shape[0]
    return pl.pallas_call(
        _post_body,
        grid=(n // BLK,),
        in_specs=[
            pl.BlockSpec((8, 128), lambda i: (0, 0)),
            pl.BlockSpec((BLK, C), lambda i: (i, 0)),
            pl.BlockSpec((BLK, C), lambda i: (i, 0)),
            pl.BlockSpec((BLK, 128), lambda i: (i, 0)),
            pl.BlockSpec((BLK, 2 * C), lambda i: (i, 0)),
            pl.BlockSpec((C, 2 * C), lambda i: (0, 0)),
            pl.BlockSpec((C, C), lambda i: (0, 0)),
        ],
        out_specs=pl.BlockSpec((BLK, C), lambda i: (i, 0)),
        out_shape=jax.ShapeDtypeStruct((n, C), jnp.float32),
    )(wvec, x, agg, deg2d, pre, Wgn_cat, W_gin)


def kernel(x, edge_index, weights, W_gcn, W_sage_self, W_sage_neigh, W_gin):
    src = edge_index[0]
    dst = edge_index[1]
    agg, deg_blk = _sc_aggregate(x, src, dst)
    pre = _dense_pre(x, jnp.concatenate([W_sage_self, W_gin], axis=1))
    deg = deg_blk.reshape(NW, 8 * C)[:, :R].reshape(PAD_N)
    deg2d = jnp.broadcast_to(deg[:N_NODES, None], (N_NODES, 128))
    wvec = jnp.pad(jnp.broadcast_to(weights.reshape(5, 1), (5, 128)),
                   ((0, 3), (0, 0)))
    return _dense_post(x, agg[:N_NODES], deg2d, pre, wvec,
                       jnp.concatenate([W_gcn, W_sage_neigh], axis=1), W_gin)


# speedup vs baseline: 1.6477x; 1.1270x over previous
import functools

import jax
import jax.numpy as jnp
from jax import lax
from jax.experimental import pallas as pl
from jax.experimental.pallas import tpu as pltpu
from jax.experimental.pallas import tpu_sc as plsc

N_NODES = 10000
C = 256
NUM_CORES = 2
NUM_SUBCORES = 16
NW = NUM_CORES * NUM_SUBCORES
R = 320
PAD_N = NW * R
STRIP = 6400
GK = 32


def _sc_aggregate(x, src, dst):
    E = src.shape[0]
    n_strips = E // STRIP
    n_cchunks = STRIP // 16

    mesh = plsc.VectorSubcoreMesh(
        core_axis_name="c", subcore_axis_name="s",
        num_cores=NUM_CORES, num_subcores=NUM_SUBCORES)

    @functools.partial(
        pl.kernel,
        out_type=(jax.ShapeDtypeStruct((PAD_N, C), jnp.float32),
                  jax.ShapeDtypeStruct((NW * 8, C), jnp.float32)),
        mesh=mesh,
        compiler_params=pltpu.CompilerParams(needs_layout_passes=False),
        scratch_types=[
            pltpu.VMEM((STRIP,), jnp.int32),
            pltpu.VMEM((STRIP,), jnp.int32),
            pltpu.VMEM((STRIP + 32,), jnp.int32),
            pltpu.VMEM((STRIP + 32,), jnp.int32),
            pltpu.VMEM((GK, C), jnp.float32),
            pltpu.VMEM((GK, C), jnp.float32),
            pltpu.SemaphoreType.DMA,
            pltpu.SemaphoreType.DMA,
            pltpu.VMEM((R, C), jnp.float32),
            pltpu.VMEM((R + 16,), jnp.float32),
            pltpu.VMEM((8, C), jnp.float32),
        ],
    )
    def k(x_hbm, src_hbm, dst_hbm, agg_out, deg_out,
          ssrc_v, sdst_v, psrc_v, pdl_v, rows_a, rows_b, sem_a, sem_b,
          acc_v, deg_v, dst_stage_v):
        cid = lax.axis_index("c")
        sid = lax.axis_index("s")
        wid = cid * NUM_SUBCORES + sid
        base = wid * R

        zero16 = jnp.zeros((16,), jnp.float32)
        one16 = jnp.ones((16,), jnp.float32)
        lane0 = lax.iota(jnp.int32, 16) == 0

        @pl.loop(0, R)
        def _(i):
            for j in range(C // 16):
                acc_v[i, pl.ds(j * 16, 16)] = zero16

        @pl.loop(0, (R + 16) // 16)
        def _(i):
            deg_v[pl.ds(i * 16, 16)] = zero16

        @pl.loop(0, (STRIP + 32) // 16)
        def _(i):
            psrc_v[pl.ds(i * 16, 16)] = jnp.zeros((16,), jnp.int32)

        @pl.loop(0, n_strips)
        def _(s):
            e0 = s * STRIP
            pltpu.sync_copy(src_hbm.at[pl.ds(e0, STRIP)], ssrc_v)
            pltpu.sync_copy(dst_hbm.at[pl.ds(e0, STRIP)], sdst_v)

            def compress(j, off):
                dv = sdst_v[pl.ds(j * 16, 16)]
                sv = ssrc_v[pl.ds(j * 16, 16)]
                rel = dv - base
                m = rel.astype(jnp.uint32) < R
                plsc.store_compressed(psrc_v.at[pl.ds(off, 16)], sv, mask=m)
                plsc.store_compressed(pdl_v.at[pl.ds(off, 16)], rel, mask=m)
                return off + plsc.all_reduce_population_count(m)[0]

            kk = plsc.parallel_loop(0, n_cchunks, carry=jnp.int32(0))(compress)

            nch = (kk + GK - 1) // GK

            def fire(c, buf, sem):
                pltpu.make_async_copy(
                    x_hbm.at[psrc_v.at[pl.ds(c * GK, GK)]], buf, sem).start()

            def drain(c, buf, sem):
                pltpu.make_async_copy(
                    x_hbm.at[psrc_v.at[pl.ds(c * GK, GK)]], buf, sem).wait()
                nrows = jnp.minimum(GK, kk - c * GK)

                @plsc.parallel_loop(0, nrows)
                def _(r):
                    dl = pdl_v[pl.ds(c * GK + r, 16)][0]
                    for j in range(C // 16):
                        plsc.addupdate(acc_v.at[dl, pl.ds(j * 16, 16)],
                                       buf[r, pl.ds(j * 16, 16)])
                    plsc.addupdate_scatter(
                        deg_v, [jnp.full((16,), dl, jnp.int32)], one16,
                        mask=lane0)

            @pl.when(nch > 0)
            def _():
                fire(0, rows_a, sem_a)

            @pl.loop(0, (nch + 1) // 2)
            def _(pair):
                c0 = 2 * pair
                c1 = c0 + 1

                @pl.when(c1 < nch)
                def _():
                    fire(c1, rows_b, sem_b)

                drain(c0, rows_a, sem_a)

                @pl.when(c1 + 1 < nch)
                def _():
                    fire(c1 + 1, rows_a, sem_a)

                @pl.when(c1 < nch)
                def _():
                    drain(c1, rows_b, sem_b)

        for j in range(16):
            dst_stage_v[0, pl.ds(j * 16, 16)] = deg_v[pl.ds(j * 16, 16)]
        for j in range(4):
            dst_stage_v[1, pl.ds(j * 16, 16)] = deg_v[pl.ds(256 + j * 16, 16)]
        pltpu.sync_copy(acc_v, agg_out.at[pl.ds(base, R)])
        pltpu.sync_copy(dst_stage_v, deg_out.at[pl.ds(wid * 8, 8)])

    return k(x, src, dst)


BLK = 1000


def _pre_body(x_ref, w_ref, o_ref):
    o_ref[...] = jnp.dot(x_ref[...], w_ref[...],
                         preferred_element_type=jnp.float32)


def _dense_pre(x, W_cat):
    n = x.shape[0]
    return pl.pallas_call(
        _pre_body,
        grid=(n // BLK,),
        in_specs=[
            pl.BlockSpec((BLK, C), lambda i: (i, 0)),
            pl.BlockSpec((C, 2 * C), lambda i: (0, 0)),
        ],
        out_specs=pl.BlockSpec((BLK, 2 * C), lambda i: (i, 0)),
        out_shape=jax.ShapeDtypeStruct((n, 2 * C), jnp.float32),
    )(x, W_cat)


def _post_body(w_ref, x_ref, a_ref, d_ref, p_ref, wgn_ref, wi_ref, o_ref):
    x_b = x_ref[...]
    s_b = a_ref[...]
    deg = jnp.maximum(d_ref[...][:, 0:1], 1.0)
    m_b = s_b / deg
    w1 = w_ref[1, 0]
    w2 = w_ref[2, 0]
    w3 = w_ref[3, 0]
    w4 = w_ref[4, 0]
    f32 = jnp.float32
    gs = jnp.dot(m_b, wgn_ref[...], preferred_element_type=f32)
    gcn = jax.nn.relu(gs[:, :C])
    sage = jax.nn.relu(p_ref[:, :C] + gs[:, C:])
    gin = jax.nn.relu(p_ref[:, C:]
                      + jnp.dot(s_b, wi_ref[...], preferred_element_type=f32))
    o_ref[...] = w1 * x_b + w2 * gcn + w3 * sage + w4 * gin


def _dense_post(x, agg, deg2d, pre, wvec, Wgn_cat, W_gin):
    n = x.shape[0]
    return pl.pallas_call(
        _post_body,
        grid=(n // BLK,),
        in_specs=[
            pl.BlockSpec((8, 128), lambda i: (0, 0)),
            pl.BlockSpec((BLK, C), lambda i: (i, 0)),
            pl.BlockSpec((BLK, C), lambda i: (i, 0)),
            pl.BlockSpec((BLK, 128), lambda i: (i, 0)),
            pl.BlockSpec((BLK, 2 * C), lambda i: (i, 0)),
            pl.BlockSpec((C, 2 * C), lambda i: (0, 0)),
            pl.BlockSpec((C, C), lambda i: (0, 0)),
        ],
        out_specs=pl.BlockSpec((BLK, C), lambda i: (i, 0)),
        out_shape=jax.ShapeDtypeStruct((n, C), jnp.float32),
    )(wvec, x, agg, deg2d, pre, Wgn_cat, W_gin)


def kernel(x, edge_index, weights, W_gcn, W_sage_self, W_sage_neigh, W_gin):
    src = edge_index[0]
    dst = edge_index[1]
    agg, deg_blk = _sc_aggregate(x, src, dst)
    pre = _dense_pre(x, jnp.concatenate([W_sage_self, W_gin], axis=1))
    deg = deg_blk.reshape(NW, 8 * C)[:, :R].reshape(PAD_N)
    deg2d = jnp.broadcast_to(deg[:N_NODES, None], (N_NODES, 128))
    wvec = jnp.pad(jnp.broadcast_to(weights.reshape(5, 1), (5, 128)),
                   ((0, 3), (0, 0)))
    return _dense_post(x, agg[:N_NODES], deg2d, pre, wvec,
                       jnp.concatenate([W_gcn, W_sage_neigh], axis=1), W_gin)
